# baseline (speedup 1.0000x reference)
import functools

import jax
import jax.numpy as jnp
from jax import lax
from jax.experimental import pallas as pl
from jax.experimental.pallas import tpu as pltpu
from jax.experimental.pallas import tpu_sc as plsc

_VOCAB = 100000
_D = 64
_B = 1024
_MAX_NORM = 1.0
_LANES = 16

_NC = 2
_NS = 16
_NW = _NC * _NS
_B_PER_W = _B // _NW

_BLK_V = 4096


_NBUF = 12


def _gather_body(tablet_hbm, idx_hbm, x_hbm, idx_v, rows_v, *bufs_sems):
    bufs = bufs_sems[:_NBUF]
    sems = bufs_sems[_NBUF:]
    wid = lax.axis_index("s") * _NC + lax.axis_index("c")
    base = wid * _B_PER_W
    pltpu.sync_copy(idx_hbm.at[pl.ds(base, _B_PER_W)], idx_v)

    ivecs = [idx_v[pl.ds(g * _LANES, _LANES)] for g in range(_B_PER_W // _LANES)]

    def start(j):
        r = ivecs[j // _LANES][j % _LANES]
        cb = pl.multiple_of((r // 128) * 128, 128)
        return pltpu.async_copy(
            tablet_hbm.at[:, pl.ds(cb, 128)], bufs[j % _NBUF], sems[j % _NBUF]
        )

    iota = lax.iota(jnp.int32, _LANES)
    pending = [start(j) for j in range(_NBUF)]
    for j in range(_B_PER_W):
        pending[j % _NBUF].wait()
        c = ivecs[j // _LANES][j % _LANES] % 128
        cols = jnp.full((_LANES,), c, jnp.int32)
        for k in range(_D // _LANES):
            vals = plsc.load_gather(bufs[j % _NBUF], [iota + _LANES * k, cols])
            rows_v[j, pl.ds(_LANES * k, _LANES)] = vals
        if j + _NBUF < _B_PER_W:
            pending[j % _NBUF] = start(j + _NBUF)
    pltpu.sync_copy(rows_v, x_hbm.at[pl.ds(base, _B_PER_W)])


_sc_gather = functools.partial(
    pl.kernel,
    out_type=jax.ShapeDtypeStruct((_B, _D), jnp.float32),
    mesh=plsc.VectorSubcoreMesh(core_axis_name="c", subcore_axis_name="s"),
    scratch_types=(
        [
            pltpu.VMEM((_B_PER_W,), jnp.int32),
            pltpu.VMEM((_B_PER_W, _D), jnp.float32),
        ]
        + [pltpu.VMEM((_D, 128), jnp.float32)] * _NBUF
        + [pltpu.SemaphoreType.DMA] * _NBUF
    ),
    compiler_params=pltpu.CompilerParams(
        use_tc_tiling_on_sc=True, needs_layout_passes=False
    ),
)(_gather_body)


def _proj_body(x_ref, wt_ref, b_ref, out_ref, xst_ref):
    @pl.when(pl.program_id(0) == 0)
    def _():
        x = x_ref[...]
        ss = jnp.sum(x * x, axis=1, keepdims=True)
        norms = jnp.sqrt(ss)
        scale = jnp.where(norms > _MAX_NORM, _MAX_NORM / norms, 1.0)
        xst_ref[...] = (x * scale).T

    acc = lax.dot_general(
        wt_ref[...], xst_ref[...], (((0,), (0,)), ((), ())),
        preferred_element_type=jnp.float32,
    )
    out_ref[...] = acc + b_ref[...].T


def kernel(inputs_, table, W, b):
    x = _sc_gather(table.T, inputs_)

    grid = (_VOCAB + _BLK_V - 1) // _BLK_V
    b2 = b.reshape(1, _VOCAB)
    wt = W.T
    out_t = pl.pallas_call(
        _proj_body,
        grid=(grid,),
        in_specs=[
            pl.BlockSpec((_B, _D), lambda i: (0, 0)),
            pl.BlockSpec((_D, _BLK_V), lambda i: (0, i)),
            pl.BlockSpec((1, _BLK_V), lambda i: (0, i)),
        ],
        out_specs=pl.BlockSpec((_BLK_V, _B), lambda i: (i, 0)),
        out_shape=jax.ShapeDtypeStruct((_VOCAB, _B), jnp.float32),
        scratch_shapes=[pltpu.VMEM((_D, _B), jnp.float32)],
        compiler_params=pltpu.CompilerParams(
            dimension_semantics=("arbitrary",),
        ),
    )(x, wt, b2)
    return out_t.T


# speedup vs baseline: 1.0113x; 1.0113x over previous
import functools

import jax
import jax.numpy as jnp
from jax import lax
from jax.experimental import pallas as pl
from jax.experimental.pallas import tpu as pltpu
from jax.experimental.pallas import tpu_sc as plsc

_VOCAB = 100000
_D = 64
_B = 1024
_MAX_NORM = 1.0
_LANES = 16

_NC = 2
_NS = 16
_NW = _NC * _NS
_B_PER_W = _B // _NW

_BLK_V = 4096


_NBUF = 8


def _gather_body(tablet_hbm, idx_hbm, x_hbm, idx_v, rows_v, *bufs_sems):
    bufs = bufs_sems[:_NBUF]
    sems = bufs_sems[_NBUF:]
    wid = lax.axis_index("s") * _NC + lax.axis_index("c")
    base = wid * _B_PER_W
    pltpu.sync_copy(idx_hbm.at[pl.ds(base, _B_PER_W)], idx_v)

    ivecs = [idx_v[pl.ds(g * _LANES, _LANES)] for g in range(_B_PER_W // _LANES)]

    def start(j):
        r = ivecs[j // _LANES][j % _LANES]
        cb = pl.multiple_of((r // 128) * 128, 128)
        return pltpu.async_copy(
            tablet_hbm.at[:, pl.ds(cb, 128)], bufs[j % _NBUF], sems[j % _NBUF]
        )

    iota = lax.iota(jnp.int32, _LANES)
    pending = [start(j) for j in range(_NBUF)]
    for j in range(_B_PER_W):
        pending[j % _NBUF].wait()
        c = ivecs[j // _LANES][j % _LANES] % 128
        cols = jnp.full((_LANES,), c, jnp.int32)
        for k in range(_D // _LANES):
            vals = plsc.load_gather(bufs[j % _NBUF], [iota + _LANES * k, cols])
            rows_v[j, pl.ds(_LANES * k, _LANES)] = vals
        if j + _NBUF < _B_PER_W:
            pending[j % _NBUF] = start(j + _NBUF)
    pltpu.sync_copy(rows_v, x_hbm.at[pl.ds(base, _B_PER_W)])


_sc_gather = functools.partial(
    pl.kernel,
    out_type=jax.ShapeDtypeStruct((_B, _D), jnp.float32),
    mesh=plsc.VectorSubcoreMesh(core_axis_name="c", subcore_axis_name="s"),
    scratch_types=(
        [
            pltpu.VMEM((_B_PER_W,), jnp.int32),
            pltpu.VMEM((_B_PER_W, _D), jnp.float32),
        ]
        + [pltpu.VMEM((_D, 128), jnp.float32)] * _NBUF
        + [pltpu.SemaphoreType.DMA] * _NBUF
    ),
    compiler_params=pltpu.CompilerParams(
        use_tc_tiling_on_sc=True, needs_layout_passes=False
    ),
)(_gather_body)


def _proj_body(x_ref, wt_ref, b_ref, out_ref, xst_ref):
    @pl.when(pl.program_id(0) == 0)
    def _():
        x = x_ref[...]
        ss = jnp.sum(x * x, axis=1, keepdims=True)
        norms = jnp.sqrt(ss)
        scale = jnp.where(norms > _MAX_NORM, _MAX_NORM / norms, 1.0)
        xst_ref[...] = (x * scale).T

    acc = lax.dot_general(
        wt_ref[...], xst_ref[...], (((0,), (0,)), ((), ())),
        preferred_element_type=jnp.float32,
    )
    out_ref[...] = acc + b_ref[...].T


def kernel(inputs_, table, W, b):
    x = _sc_gather(table.T, inputs_)

    grid = (_VOCAB + _BLK_V - 1) // _BLK_V
    b2 = b.reshape(1, _VOCAB)
    wt = W.T
    out_t = pl.pallas_call(
        _proj_body,
        grid=(grid,),
        in_specs=[
            pl.BlockSpec((_B, _D), lambda i: (0, 0)),
            pl.BlockSpec((_D, _BLK_V), lambda i: (0, i)),
            pl.BlockSpec((1, _BLK_V), lambda i: (0, i)),
        ],
        out_specs=pl.BlockSpec((_BLK_V, _B), lambda i: (i, 0)),
        out_shape=jax.ShapeDtypeStruct((_VOCAB, _B), jnp.float32),
        scratch_shapes=[pltpu.VMEM((_D, _B), jnp.float32)],
        compiler_params=pltpu.CompilerParams(
            dimension_semantics=("arbitrary",),
        ),
    )(x, wt, b2)
    return out_t.T
